# baseline (speedup 1.0000x reference)
import functools

import jax
import jax.numpy as jnp
from jax import lax
from jax.experimental import pallas as pl
from jax.experimental.pallas import tpu as pltpu
from jax.experimental.pallas import tpu_sc as plsc

N_EMB = 1000000
EMB_DIM = 32
SCALE = 10.0
LANES = 16

NUM_CORES = 2
NUM_SUBCORES = 16
NW = NUM_CORES * NUM_SUBCORES

XROWS = 16384
XCOLS = 100
XPAD = 128
ROWS_PER_W = XROWS // NW
RCHUNK = 8
SLOTS = RCHUNK * XPAD
OROW = RCHUNK * XCOLS // 4
NCHUNK = ROWS_PER_W // RCHUNK
B = XROWS * XCOLS

_mesh = plsc.VectorSubcoreMesh(core_axis_name="c", subcore_axis_name="s")


@functools.partial(
    pl.kernel,
    mesh=_mesh,
    out_type=jax.ShapeDtypeStruct((B // 4, 128), jnp.float32),
    scratch_types=[
        pltpu.VMEM((SLOTS,), jnp.int32),
        pltpu.VMEM((SLOTS, EMB_DIM), jnp.float32),
        pltpu.VMEM((OROW, 128), jnp.float32),
        pltpu.SemaphoreType.DMA,
    ],
    compiler_params=pltpu.CompilerParams(use_tc_tiling_on_sc=False),
)
def _scaled_gather(x_hbm, tab_hbm, out_hbm, idx_v, g_v, o_v, sem):
    wid = lax.axis_index("s") * NUM_CORES + lax.axis_index("c")
    row_base = wid * ROWS_PER_W

    def chunk_body(g, carry):
        r0 = row_base + g * RCHUNK
        pltpu.sync_copy(x_hbm.at[pl.ds(r0 * XPAD, SLOTS)], idx_v)
        pltpu.async_copy(tab_hbm.at[idx_v], g_v, sem).wait()

        def row_body(xr, c1):
            def quad_body(m, c2):
                for q in range(4):
                    for h in range(EMB_DIM // LANES):
                        src = g_v[xr * XPAD + 4 * m + q,
                                  pl.ds(h * LANES, LANES)]
                        o_v[xr * (XCOLS // 4) + m,
                            pl.ds(q * EMB_DIM + h * LANES, LANES)] = src * SCALE
                return c2

            return lax.fori_loop(0, XCOLS // 4, quad_body, c1)

        lax.fori_loop(0, RCHUNK, row_body, 0)
        pltpu.sync_copy(o_v, out_hbm.at[pl.ds(r0 * (XCOLS // 4), OROW)])
        return carry

    lax.fori_loop(0, NCHUNK, chunk_body, 0)


def kernel(x, table):
    xp = jnp.pad(x, ((0, 0), (0, XPAD - XCOLS))).reshape(-1)
    out = _scaled_gather(xp, table)
    return out.reshape(x.shape[0], x.shape[1], EMB_DIM)


# speedup vs baseline: 2.8875x; 2.8875x over previous
import functools

import jax
import jax.numpy as jnp
from jax import lax
from jax.experimental import pallas as pl
from jax.experimental.pallas import tpu as pltpu
from jax.experimental.pallas import tpu_sc as plsc

N_EMB = 1000000
EMB_DIM = 32
SCALE = 10.0
LANES = 16

NUM_CORES = 2
NUM_SUBCORES = 16
NW = NUM_CORES * NUM_SUBCORES

XROWS = 16384
XCOLS = 100
XPAD = 128
ROWS_PER_W = XROWS // NW
RCHUNK = 8
SLOTS = RCHUNK * XPAD
OROW = RCHUNK * XCOLS // 4
NCHUNK = ROWS_PER_W // RCHUNK
B = XROWS * XCOLS

_mesh = plsc.VectorSubcoreMesh(core_axis_name="c", subcore_axis_name="s")


@functools.partial(
    pl.kernel,
    mesh=_mesh,
    out_type=jax.ShapeDtypeStruct((B // 4, 128), jnp.float32),
    scratch_types=[
        pltpu.VMEM((SLOTS,), jnp.int32),
        pltpu.VMEM((SLOTS, EMB_DIM), jnp.float32),
        pltpu.VMEM((OROW, 128), jnp.float32),
        pltpu.SemaphoreType.DMA,
    ],
    compiler_params=pltpu.CompilerParams(use_tc_tiling_on_sc=False),
)
def _scaled_gather(x_hbm, tab_hbm, out_hbm, idx_v, g_v, o_v, sem):
    wid = lax.axis_index("s") * NUM_CORES + lax.axis_index("c")
    row_base = wid * ROWS_PER_W

    def chunk_body(g, carry):
        r0 = row_base + g * RCHUNK
        pltpu.sync_copy(x_hbm.at[pl.ds(r0 * XPAD, SLOTS)], idx_v)
        pltpu.async_copy(tab_hbm.at[idx_v], g_v, sem).wait()

        def row_body(xr, c1):
            def quad_body(m, c2):
                for q in range(4):
                    for h in range(EMB_DIM // LANES):
                        src = g_v[xr * XPAD + 4 * m + q,
                                  pl.ds(h * LANES, LANES)]
                        o_v[xr * (XCOLS // 4) + m,
                            pl.ds(q * EMB_DIM + h * LANES, LANES)] = src * SCALE
                return c2

            return lax.fori_loop(0, XCOLS // 4, quad_body, c1)

        lax.fori_loop(0, RCHUNK, row_body, 0)
        pltpu.sync_copy(o_v, out_hbm.at[pl.ds(r0 * (XCOLS // 4), OROW)])
        return carry

    lax.fori_loop(0, NCHUNK, chunk_body, 0)


def kernel(x, table):
    fill = (jnp.arange(XROWS * (XPAD - XCOLS), dtype=x.dtype) * 131) % N_EMB
    xp = jnp.concatenate(
        [x, fill.reshape(XROWS, XPAD - XCOLS)], axis=1
    ).reshape(-1)
    out = _scaled_gather(xp, table)
    return out.reshape(x.shape[0], x.shape[1], EMB_DIM)


# speedup vs baseline: 3.7378x; 1.2945x over previous
import functools

import jax
import jax.numpy as jnp
from jax import lax
from jax.experimental import pallas as pl
from jax.experimental.pallas import tpu as pltpu
from jax.experimental.pallas import tpu_sc as plsc

N_EMB = 1000000
EMB_DIM = 32
SCALE = 10.0
LANES = 16

NUM_CORES = 2
NUM_SUBCORES = 16
NW = NUM_CORES * NUM_SUBCORES

XROWS = 16384
XCOLS = 100
ROWS_PER_W = XROWS // NW
RCHUNK = 8
LOOK = RCHUNK * XCOLS
OROW = LOOK // 4
NCHUNK = ROWS_PER_W // RCHUNK
B = XROWS * XCOLS

_mesh = plsc.VectorSubcoreMesh(core_axis_name="c", subcore_axis_name="s")


@functools.partial(
    pl.kernel,
    mesh=_mesh,
    out_type=jax.ShapeDtypeStruct((B // 4, 128), jnp.float32),
    scratch_types=[
        pltpu.VMEM((RCHUNK, XCOLS), jnp.int32),
        pltpu.VMEM((LOOK,), jnp.int32),
        pltpu.VMEM((LOOK, EMB_DIM), jnp.float32),
        pltpu.VMEM((OROW, 128), jnp.float32),
        pltpu.SemaphoreType.DMA,
    ],
    compiler_params=pltpu.CompilerParams(use_tc_tiling_on_sc=False),
)
def _scaled_gather(x_hbm, tab_hbm, out_hbm, x8_v, idx_v, g_v, o_v, sem):
    wid = lax.axis_index("s") * NUM_CORES + lax.axis_index("c")
    row_base = wid * ROWS_PER_W

    def chunk_body(g, carry):
        r0 = row_base + g * RCHUNK
        pltpu.sync_copy(x_hbm.at[pl.ds(r0, RCHUNK)], x8_v)

        for xr in range(RCHUNK):
            for c in (0, 16, 32, 48, 64, 80, 84):
                idx_v[pl.ds(xr * XCOLS + c, LANES)] = x8_v[xr, pl.ds(c, LANES)]

        pltpu.async_copy(tab_hbm.at[idx_v], g_v, sem).wait()

        def repack(r, c1):
            for q in range(4):
                for h in range(EMB_DIM // LANES):
                    src = g_v[4 * r + q, pl.ds(h * LANES, LANES)]
                    o_v[r, pl.ds(q * EMB_DIM + h * LANES, LANES)] = src * SCALE
            return c1

        lax.fori_loop(0, OROW, repack, 0)
        pltpu.sync_copy(o_v, out_hbm.at[pl.ds(r0 * (XCOLS // 4), OROW)])
        return carry

    lax.fori_loop(0, NCHUNK, chunk_body, 0)


def kernel(x, table):
    out = _scaled_gather(x, table)
    return out.reshape(x.shape[0], x.shape[1], EMB_DIM)
